# baseline (speedup 1.0000x reference)
import functools

import jax
import jax.numpy as jnp
from jax import lax
from jax.experimental import pallas as pl
from jax.experimental.pallas import tpu as pltpu
from jax.experimental.pallas import tpu_sc as plsc

NUM_SPK = 100000
EMBD_DIM = 64
BATCH = 16384

_NC = 2
_NS = 16
_NW = _NC * _NS
_BPW = BATCH // _NW

_mesh = plsc.VectorSubcoreMesh(core_axis_name="c", subcore_axis_name="s")


@functools.partial(
    pl.kernel,
    mesh=_mesh,
    out_type=jax.ShapeDtypeStruct((BATCH, EMBD_DIM), jnp.float32),
    scratch_types=[
        pltpu.VMEM((_BPW,), jnp.int32),
        pltpu.VMEM((_BPW, EMBD_DIM), jnp.float32),
        pltpu.SemaphoreType.DMA,
    ],
    compiler_params=pltpu.CompilerParams(
        use_tc_tiling_on_sc=True, needs_layout_passes=False
    ),
)
def _gather_kernel(idx_hbm, table_hbm, out_hbm, idx_v, rows_v, sem_g):
    wid = lax.axis_index("s") * _NC + lax.axis_index("c")
    base = wid * _BPW
    pltpu.sync_copy(idx_hbm.at[pl.ds(base, _BPW)], idx_v)

    def _row_body(g, carry):
        svec = idx_v[pl.ds(g * 16, 16)]
        for j in range(16):
            s = svec[j]
            pltpu.async_copy(
                table_hbm.at[pl.ds(s, 1)],
                rows_v.at[pl.ds(g * 16 + j, 1)],
                sem_g,
            )
        return carry

    lax.fori_loop(0, _BPW // 16, _row_body, 0)

    pltpu.make_async_copy(
        table_hbm.at[pl.ds(0, _BPW)], rows_v, sem_g
    ).wait()

    pltpu.sync_copy(rows_v, out_hbm.at[pl.ds(base, _BPW)])


def kernel(spk_inds, table):
    table = lax.optimization_barrier(table)
    return _gather_kernel(spk_inds.astype(jnp.int32), table)
